# Initial kernel scaffold; baseline (speedup 1.0000x reference)
import jax
import jax.numpy as jnp
from jax.experimental import pallas as pl


def kernel(a, b):
    raise NotImplementedError("write your pallas kernel here")


# speedup vs baseline: 13.1283x; 13.1283x over previous
import functools

import numpy as np
import jax
import jax.numpy as jnp
from jax import lax
from jax.experimental import pallas as pl
from jax.experimental.pallas import tpu as pltpu
from jax.experimental.pallas import tpu_sc as plsc

D = 8
NBLADES = 256
BATCH = 1024
NTILES = 32
COLS = BATCH // NTILES
_METRIC = [1] * 7 + [-1]


def _popcount(x: int) -> int:
    return bin(x).count("1")


def _reorder_sign(p: int, q: int) -> int:
    c = 0
    for i in range(D):
        if (p >> i) & 1:
            for j in range(D):
                if (q >> j) & 1 and i > j:
                    c += 1
    return -1 if (c & 1) else 1


_ORDER = sorted(
    range(NBLADES),
    key=lambda m: (_popcount(m), tuple(i for i in range(D) if (m >> i) & 1)),
)
_M2I = np.zeros(NBLADES, dtype=np.int32)
for _i, _m in enumerate(_ORDER):
    _M2I[_m] = _i

_S3 = [[_reorder_sign(p, q) for q in range(8)] for p in range(8)]
_SH = np.zeros((32, 32), dtype=np.float32)
_CHI = np.zeros((32,), dtype=np.float32)
for _ph in range(32):
    _CHI[_ph] = -1.0 if (_popcount(_ph) & 1) else 1.0
    for _qh in range(32):
        _s = _reorder_sign(_ph << 3, _qh << 3)
        if (_ph & 16) and (_qh & 16):
            _s = -_s
        _SH[_ph, _qh] = float(_s)

_J3 = [[p ^ c for p in range(8)] for c in range(8)]

_SGNV = np.zeros((32, 32, 2, 16), dtype=np.float32)
for _ch in range(32):
    for _ph in range(32):
        _s0 = _SH[_ph, _ph ^ _ch]
        _SGNV[_ch, _ph, 0, :] = _s0
        _SGNV[_ch, _ph, 1, :] = _s0 * _CHI[_ph]
_SGNV = _SGNV.reshape(2048, 16)

_mesh = plsc.VectorSubcoreMesh(core_axis_name="c", subcore_axis_name="s")


@functools.partial(
    pl.kernel,
    out_type=jax.ShapeDtypeStruct((NTILES * NBLADES, COLS), jnp.float32),
    mesh=_mesh,
    scratch_types=[
        pltpu.VMEM((2, 128), jnp.int32),
        pltpu.VMEM((NBLADES, COLS), jnp.float32),
        pltpu.VMEM((NBLADES, COLS), jnp.float32),
        pltpu.VMEM((NBLADES, COLS), jnp.float32),
        pltpu.VMEM((2048, 16), jnp.float32),
        pltpu.SemaphoreType.DMA,
    ],
    compiler_params=pltpu.CompilerParams(use_tc_tiling_on_sc=False),
)
def _gp_sc(a_hbm, b_hbm, idx_hbm, sgn_hbm, out_hbm,
           idx_v, a_v, b_v, o_v, sgn_v, sem):
    wid = lax.axis_index("s") * 2 + lax.axis_index("c")
    base = wid * NBLADES

    pltpu.sync_copy(idx_hbm, idx_v)
    pltpu.sync_copy(sgn_hbm, sgn_v)

    for j in range(2):
        for k in range(8):
            sl = idx_v[j, pl.ds(k * 16, 16)]
            idx_v[j, pl.ds(k * 16, 16)] = sl + base

    cps = []
    for j in range(2):
        cps.append(pltpu.async_copy(
            a_hbm.at[idx_v.at[j]], a_v.at[pl.ds(j * 128, 128)], sem))
        cps.append(pltpu.async_copy(
            b_hbm.at[idx_v.at[j]], b_v.at[pl.ds(j * 128, 128)], sem))
    for cp in cps:
        cp.wait()

    for g in range(2):
        col = g * 16

        def ch_body(ch, carry, col=col):
            def ph_body(ph, accs, ch=ch, col=col):
                qh = ph ^ ch
                srow = (ch * 32 + ph) * 2
                sv0 = sgn_v[srow]
                sv1 = sgn_v[srow + 1]
                arow = ph * 8
                brow = qh * 8
                bs = []
                for j in range(8):
                    bj = b_v[brow + j, pl.ds(col, 16)]
                    bs.append(bj * (sv0 if (_popcount(j) & 1) == 0 else sv1))
                avs = [a_v[arow + p, pl.ds(col, 16)] for p in range(8)]
                new = list(accs)
                for cl in range(8):
                    row3 = _S3
                    for p in range(8):
                        ql = _J3[cl][p]
                        t = avs[p] * bs[ql]
                        if row3[p][ql] > 0:
                            new[cl] = new[cl] + t
                        else:
                            new[cl] = new[cl] - t
                return tuple(new)

            accs0 = tuple(jnp.zeros((16,), jnp.float32) for _ in range(8))
            accs = lax.fori_loop(0, 32, ph_body, accs0)
            for cl in range(8):
                o_v[ch * 8 + cl, pl.ds(col, 16)] = accs[cl]
            return carry

        lax.fori_loop(0, 32, ch_body, 0)

    cps = []
    for j in range(2):
        cps.append(pltpu.async_copy(
            o_v.at[pl.ds(j * 128, 128)], out_hbm.at[idx_v.at[j]], sem))
    for cp in cps:
        cp.wait()


_IDX_CONST = _M2I.reshape(2, 128)


def kernel(a, b):
    at = a.T.reshape(NBLADES, NTILES, COLS).transpose(1, 0, 2)
    bt = b.T.reshape(NBLADES, NTILES, COLS).transpose(1, 0, 2)
    at = at.reshape(NTILES * NBLADES, COLS)
    bt = bt.reshape(NTILES * NBLADES, COLS)
    out = _gp_sc(at, bt, _IDX_CONST, _SGNV)
    res = out.reshape(NTILES, NBLADES, COLS).transpose(1, 0, 2)
    return res.reshape(NBLADES, BATCH).T
